# baseline (speedup 1.0000x reference)
import jax
import jax.numpy as jnp
from jax import lax
from jax.experimental import pallas as pl
from jax.experimental.pallas import tpu as pltpu
from jax.experimental.pallas import tpu_sc as plsc

B = 4
S = 4096
D = 1024
NW = 32
ROWS_PER_W = (B * S) // NW
R = 32
N_ROUND = ROWS_PER_W // R
RD = R * D
NS = 16
SP_WORDS = NS * RD


def _sc_body(x_hbm, out_hbm, sp, in_sem, out_sem):
    cid = lax.axis_index("c")
    sid = lax.axis_index("s")
    wid = sid * 2 + cid
    base = wid * ROWS_PER_W * D
    my_off = sid * RD

    def start_in(r, p):
        pltpu.async_copy(x_hbm.at[pl.ds(base + r * RD, RD)],
                         sp.at[p, pl.ds(my_off, RD)], in_sem.at[p])

    def wait_in(r, p):
        pltpu.make_async_copy(x_hbm.at[pl.ds(base + r * RD, RD)],
                              sp.at[p, pl.ds(my_off, RD)], in_sem.at[p]).wait()

    def start_out(r, p):
        pltpu.async_copy(sp.at[p, pl.ds(my_off, RD)],
                         out_hbm.at[pl.ds(base + r * RD, RD)], out_sem.at[p])

    def wait_out(r, p):
        pltpu.make_async_copy(sp.at[p, pl.ds(my_off, RD)],
                              out_hbm.at[pl.ds(base + r * RD, RD)],
                              out_sem.at[p]).wait()

    start_in(0, 0)
    start_in(1, 1)

    @pl.loop(0, N_ROUND, step=2)
    def round_loop(r0):
        for p in range(2):
            r = r0 + p
            wait_in(r, p)
            start_out(r, p)

            @pl.when(r + 2 < N_ROUND)
            def _():
                wait_out(r, p)
                start_in(r + 2, p)

    wait_out(N_ROUND - 2, 0)
    wait_out(N_ROUND - 1, 1)


@jax.jit
def kernel(x, emb):
    mesh = plsc.VectorSubcoreMesh(core_axis_name="c", subcore_axis_name="s")
    k = pl.kernel(
        _sc_body,
        out_type=jax.ShapeDtypeStruct((B * S * D,), jnp.float32),
        mesh=mesh,
        scratch_types=[
            pltpu.VMEM_SHARED((2, SP_WORDS), jnp.float32),
            pltpu.SemaphoreType.DMA((2,)),
            pltpu.SemaphoreType.DMA((2,)),
        ],
    )
    out = k(x.reshape(-1))
    return out.reshape(B, S, D)


# speedup vs baseline: 3.8651x; 3.8651x over previous
import functools

import jax
import jax.numpy as jnp
from jax.experimental import pallas as pl
from jax.experimental.pallas import tpu as pltpu

B = 4
S = 4096
D = 1024
S_BLK = 2048


def _add_body(x_ref, emb_ref, out_ref):
    out_ref[...] = x_ref[...] + emb_ref[...][None]


@jax.jit
def kernel(x, emb):
    n_s = S // S_BLK
    grid = (n_s, B)
    out = pl.pallas_call(
        _add_body,
        grid=grid,
        in_specs=[
            pl.BlockSpec((1, S_BLK, D), lambda s, b: (b, s, 0)),
            pl.BlockSpec((S_BLK, D), lambda s, b: (s, 0)),
        ],
        out_specs=pl.BlockSpec((1, S_BLK, D), lambda s, b: (b, s, 0)),
        out_shape=jax.ShapeDtypeStruct((B, S, D), jnp.float32),
        compiler_params=pltpu.CompilerParams(
            dimension_semantics=("arbitrary", "arbitrary"),
        ),
    )(x, emb)
    return out


# speedup vs baseline: 3.8686x; 1.0009x over previous
import jax
import jax.numpy as jnp
from jax.experimental import pallas as pl
from jax.experimental.pallas import tpu as pltpu

B = 4
S = 4096
D = 1024
CH = 1024
N_C = S // CH
T = N_C * B
NBUF = 4


def _body(x_hbm, emb_hbm, out_hbm, x_bufs, emb_bufs, in_sem, out_sem, emb_sem):
    def start_in(t):
        c, b, buf = t // B, t % B, t % NBUF
        pltpu.make_async_copy(x_hbm.at[b, pl.ds(c * CH, CH)], x_bufs.at[buf],
                              in_sem.at[buf]).start()

    def wait_in(t):
        c, b, buf = t // B, t % B, t % NBUF
        pltpu.make_async_copy(x_hbm.at[b, pl.ds(c * CH, CH)], x_bufs.at[buf],
                              in_sem.at[buf]).wait()

    def start_out(t):
        c, b, buf = t // B, t % B, t % NBUF
        pltpu.make_async_copy(x_bufs.at[buf], out_hbm.at[b, pl.ds(c * CH, CH)],
                              out_sem.at[buf]).start()

    def wait_out(t):
        c, b, buf = t // B, t % B, t % NBUF
        pltpu.make_async_copy(x_bufs.at[buf], out_hbm.at[b, pl.ds(c * CH, CH)],
                              out_sem.at[buf]).wait()

    def start_emb(c):
        pltpu.make_async_copy(emb_hbm.at[pl.ds(c * CH, CH)], emb_bufs.at[c % 2],
                              emb_sem.at[c % 2]).start()

    def wait_emb(c):
        pltpu.make_async_copy(emb_hbm.at[pl.ds(c * CH, CH)], emb_bufs.at[c % 2],
                              emb_sem.at[c % 2]).wait()

    start_emb(0)
    start_in(0)
    start_in(1)

    for t in range(T):
        c, b, buf = t // B, t % B, t % NBUF
        if t >= 2:
            wait_out(t - 2)
        if t + 2 < T:
            start_in(t + 2)
        if b == 0:
            if c + 1 < N_C:
                start_emb(c + 1)
            wait_emb(c)
        wait_in(t)
        x_bufs[buf] = x_bufs[buf] + emb_bufs[c % 2]
        start_out(t)

    wait_out(T - 2)
    wait_out(T - 1)


@jax.jit
def kernel(x, emb):
    out = pl.pallas_call(
        _body,
        in_specs=[
            pl.BlockSpec(memory_space=pl.ANY),
            pl.BlockSpec(memory_space=pl.ANY),
        ],
        out_specs=pl.BlockSpec(memory_space=pl.ANY),
        out_shape=jax.ShapeDtypeStruct((B, S, D), jnp.float32),
        scratch_shapes=[
            pltpu.VMEM((NBUF, CH, D), jnp.float32),
            pltpu.VMEM((2, CH, D), jnp.float32),
            pltpu.SemaphoreType.DMA((NBUF,)),
            pltpu.SemaphoreType.DMA((NBUF,)),
            pltpu.SemaphoreType.DMA((2,)),
        ],
    )(x, emb)
    return out


# speedup vs baseline: 3.8830x; 1.0037x over previous
import jax
import jax.numpy as jnp
from jax.experimental import pallas as pl
from jax.experimental.pallas import tpu as pltpu

B = 4
S = 4096
D = 1024
CH = 1024
N_C = S // CH
T = N_C * B
NBUF = 6


def _body(x_hbm, emb_hbm, out_hbm, x_bufs, emb_bufs, in_sem, out_sem, emb_sem):
    def start_in(t):
        c, b, buf = t // B, t % B, t % NBUF
        pltpu.make_async_copy(x_hbm.at[b, pl.ds(c * CH, CH)], x_bufs.at[buf],
                              in_sem.at[buf]).start()

    def wait_in(t):
        c, b, buf = t // B, t % B, t % NBUF
        pltpu.make_async_copy(x_hbm.at[b, pl.ds(c * CH, CH)], x_bufs.at[buf],
                              in_sem.at[buf]).wait()

    def start_out(t):
        c, b, buf = t // B, t % B, t % NBUF
        pltpu.make_async_copy(x_bufs.at[buf], out_hbm.at[b, pl.ds(c * CH, CH)],
                              out_sem.at[buf]).start()

    def wait_out(t):
        c, b, buf = t // B, t % B, t % NBUF
        pltpu.make_async_copy(x_bufs.at[buf], out_hbm.at[b, pl.ds(c * CH, CH)],
                              out_sem.at[buf]).wait()

    def start_emb(c):
        pltpu.make_async_copy(emb_hbm.at[pl.ds(c * CH, CH)], emb_bufs.at[c % 2],
                              emb_sem.at[c % 2]).start()

    def wait_emb(c):
        pltpu.make_async_copy(emb_hbm.at[pl.ds(c * CH, CH)], emb_bufs.at[c % 2],
                              emb_sem.at[c % 2]).wait()

    start_emb(0)
    start_in(0)
    start_in(1)
    start_in(2)

    for t in range(T):
        c, b, buf = t // B, t % B, t % NBUF
        if t >= 3:
            wait_out(t - 3)
        if t + 3 < T:
            start_in(t + 3)
        if b == 0:
            if c + 1 < N_C:
                start_emb(c + 1)
            wait_emb(c)
        wait_in(t)
        x_bufs[buf] = x_bufs[buf] + emb_bufs[c % 2]
        start_out(t)

    wait_out(T - 3)
    wait_out(T - 2)
    wait_out(T - 1)


@jax.jit
def kernel(x, emb):
    out = pl.pallas_call(
        _body,
        in_specs=[
            pl.BlockSpec(memory_space=pl.ANY),
            pl.BlockSpec(memory_space=pl.ANY),
        ],
        out_specs=pl.BlockSpec(memory_space=pl.ANY),
        out_shape=jax.ShapeDtypeStruct((B, S, D), jnp.float32),
        scratch_shapes=[
            pltpu.VMEM((NBUF, CH, D), jnp.float32),
            pltpu.VMEM((2, CH, D), jnp.float32),
            pltpu.SemaphoreType.DMA((NBUF,)),
            pltpu.SemaphoreType.DMA((NBUF,)),
            pltpu.SemaphoreType.DMA((2,)),
        ],
    )(x, emb)
    return out
